# baseline (speedup 1.0000x reference)
import functools

import jax
import jax.numpy as jnp
from jax import lax
from jax.experimental import pallas as pl
from jax.experimental.pallas import tpu as pltpu
from jax.experimental.pallas import tpu_sc as plsc

_NUM_CORES = 2
_NUM_SUBCORES = 16
_NUM_WORKERS = _NUM_CORES * _NUM_SUBCORES
_CHUNK = 128


@functools.lru_cache(maxsize=None)
def _make_gather_t(vocab, dim, batch):
    h_per_w = dim // _NUM_WORKERS
    n_chunks = batch // _CHUNK
    mesh = plsc.VectorSubcoreMesh(core_axis_name="c", subcore_axis_name="s")

    @functools.partial(
        pl.kernel,
        mesh=mesh,
        out_type=jax.ShapeDtypeStruct((dim, batch), jnp.float32),
        scratch_types=[
            pltpu.VMEM((batch,), jnp.int32),
            pltpu.VMEM((h_per_w, batch), jnp.float32),
            pltpu.SemaphoreType.DMA,
        ],
        compiler_params=pltpu.CompilerParams(use_tc_tiling_on_sc=False),
    )
    def gather_kernel(table_t_hbm, idx_hbm, out_hbm, idx_v, rows_v, sem):
        wid = lax.axis_index("s") * _NUM_CORES + lax.axis_index("c")
        h0 = wid * h_per_w
        pltpu.sync_copy(idx_hbm, idx_v)

        def chunk_body(c, carry):
            for j in range(h_per_w):
                pltpu.async_copy(
                    table_t_hbm.at[h0 + j].at[idx_v.at[pl.ds(c * _CHUNK, _CHUNK)]],
                    rows_v.at[j].at[pl.ds(c * _CHUNK, _CHUNK)],
                    sem,
                )
            return carry

        lax.fori_loop(0, n_chunks, chunk_body, 0)
        pltpu.make_async_copy(out_hbm.at[pl.ds(0, h_per_w)], rows_v, sem).wait()
        pltpu.sync_copy(rows_v, out_hbm.at[pl.ds(h0, h_per_w)])

    return gather_kernel


def kernel(labels, train, table):
    del train
    labels = labels.reshape(-1)
    table_t = table.T
    out_t = _make_gather_t(table.shape[0], table.shape[1], labels.shape[0])(
        table_t, labels
    )
    return out_t.T
